# baseline (speedup 1.0000x reference)
import functools

import jax
import jax.numpy as jnp
from jax.experimental import pallas as pl
from jax.experimental.pallas import tpu as pltpu

_TOKENS_PER_TILE = 2048
_LANES = 128
_CORES = 2


def _round_up(x, m):
    return (x + m - 1) // m * m


def _gather_tile_kernel(ids_ref, table_hbm, o_ref, tab_vmem, sem,
                        *, tokens, p, steps_per_core):
    core = pl.program_id(0)
    j = pl.program_id(1)

    @pl.when(j == 0)
    def _load_table():
        cp = pltpu.make_async_copy(table_hbm, tab_vmem, sem)
        cp.start()
        cp.wait()

    base = (core * steps_per_core + j) * tokens
    for mi in range(tokens):
        idx = pl.multiple_of(ids_ref[base + mi], p)
        slab = tab_vmem[pl.ds(idx, p), :]
        o_ref[mi, :] = slab.reshape(p * _LANES)


def kernel(sentence, embed_table):
    B, S = sentence.shape
    V, E = embed_table.shape
    T = _TOKENS_PER_TILE
    p = E // _LANES

    flat = sentence.reshape(-1).astype(jnp.int32)
    N = flat.shape[0]
    N_pad = _round_up(N, T * _CORES)
    if N_pad != N:
        flat = jnp.pad(flat, (0, N_pad - N))
    ids = flat * p

    table_v = embed_table.reshape(V * p, _LANES)
    steps_per_core = N_pad // (T * _CORES)
    grid = (_CORES, steps_per_core)

    vmem_bytes = V * E * 4 + 4 * T * E * 4 + (4 << 20)

    out = pl.pallas_call(
        functools.partial(_gather_tile_kernel, tokens=T, p=p,
                          steps_per_core=steps_per_core),
        out_shape=jax.ShapeDtypeStruct((N_pad, E), embed_table.dtype),
        grid_spec=pltpu.PrefetchScalarGridSpec(
            num_scalar_prefetch=1,
            grid=grid,
            in_specs=[
                pl.BlockSpec(memory_space=pl.ANY),
            ],
            out_specs=pl.BlockSpec(
                (T, E), lambda i, j, ids, spc=steps_per_core: (i * spc + j, 0)
            ),
            scratch_shapes=[
                pltpu.VMEM((V * p, _LANES), embed_table.dtype),
                pltpu.SemaphoreType.DMA,
            ],
        ),
        compiler_params=pltpu.CompilerParams(
            dimension_semantics=("parallel", "arbitrary"),
            vmem_limit_bytes=vmem_bytes,
        ),
    )(ids, table_v)

    return out[:N].reshape(B, S, E)
